# baseline (speedup 1.0000x reference)
import functools
import jax
import jax.numpy as jnp
from jax import lax
from jax.experimental import pallas as pl
from jax.experimental.pallas import tpu as pltpu
from jax.experimental.pallas import tpu_sc as plsc

_NC, _NS = 2, 16
_NW = _NC * _NS
_ROWS = 4


def _sc_body(n_, c_, h_, w_, tgt_hbm, wtab_hbm, ii_hbm, zero_hbm, out_hbm,
             tgtv0, tgtv1, obuf0, obuf1, wtabv, iiv, sem_in0, sem_in1,
             sem_out0, sem_out1):
    wid = lax.axis_index("s") * _NC + lax.axis_index("c")
    px = _ROWS * w_
    items_total = n_ * (h_ // _ROWS)
    items_per_w = items_total // _NW
    ngrp = px // 16
    hw = h_ * w_
    chw = c_ * hw

    tgts = (tgtv0, tgtv1)
    obufs = (obuf0, obuf1)
    sems_in = (sem_in0, sem_in1)
    sems_out = (sem_out0, sem_out1)

    pltpu.sync_copy(zero_hbm, obuf0)
    pltpu.sync_copy(zero_hbm, obuf1)
    pltpu.sync_copy(tgt_hbm.at[pl.ds(0, px)], tgtv0)
    pltpu.sync_copy(tgt_hbm.at[pl.ds(0, px)], tgtv1)
    pltpu.sync_copy(wtab_hbm, wtabv)
    pltpu.sync_copy(ii_hbm, iiv)

    ii_vec = iiv[...]
    lane = jax.lax.iota(jnp.int32, 16)

    def scatter_pass(tgt_ref, obuf_ref, value_from_table):
        def grp(i, _):
            t = tgt_ref[pl.ds(i * 16, 16)]
            tcl = jnp.clip(t, 0, c_ - 1)
            mask = t != ii_vec
            p = i * 16 + lane
            idx = tcl * px + p
            if value_from_table:
                val = plsc.load_gather(wtabv, [tcl])
            else:
                val = jnp.zeros((16,), jnp.float32)
            plsc.store_scatter(obuf_ref, [idx], val, mask=mask)
            return 0
        lax.fori_loop(0, ngrp, grp, 0)

    def item_pair(kk, _):
        for b in range(2):
            k = kk * 2 + b
            g = wid * items_per_w + k
            n = g // (h_ // _ROWS)
            hc = lax.rem(g, h_ // _ROWS)
            scatter_pass(tgts[b], obufs[b], value_from_table=False)
            cp_in = pltpu.async_copy(
                tgt_hbm.at[pl.ds(g * px, px)], tgts[b], sems_in[b])
            cp_in.wait()
            scatter_pass(tgts[b], obufs[b], value_from_table=True)
            base = n * chw + hc * (_ROWS * w_)
            cps = []
            for c in range(c_):
                cps.append(pltpu.async_copy(
                    obufs[b].at[pl.ds(c * px, px)],
                    out_hbm.at[pl.ds(base + c * hw, px)],
                    sems_out[b]))
            for cp in cps:
                cp.wait()
        return 0

    lax.fori_loop(0, items_per_w // 2, item_pair, 0)


def kernel(grad_output, x, target, weight, reduction, ignore_index, total_weight):
    n_, c_, h_, w_ = x.shape
    scal = jnp.where(reduction == 1, grad_output / total_weight, grad_output)
    wtab = jnp.zeros((32,), jnp.float32).at[:c_].set(
        -jnp.asarray(weight, jnp.float32) * scal)
    ii16 = jnp.full((16,), ignore_index, jnp.int32)
    px = _ROWS * w_
    zeros = jnp.zeros((c_ * px,), jnp.float32)
    tflat = target.reshape(-1)

    mesh = plsc.VectorSubcoreMesh(core_axis_name="c", subcore_axis_name="s")
    out = pl.kernel(
        functools.partial(_sc_body, n_, c_, h_, w_),
        out_type=jax.ShapeDtypeStruct((n_ * c_ * h_ * w_,), jnp.float32),
        mesh=mesh,
        scratch_types=[
            pltpu.VMEM((px,), jnp.int32),
            pltpu.VMEM((px,), jnp.int32),
            pltpu.VMEM((c_ * px,), jnp.float32),
            pltpu.VMEM((c_ * px,), jnp.float32),
            pltpu.VMEM((32,), jnp.float32),
            pltpu.VMEM((16,), jnp.int32),
            pltpu.SemaphoreType.DMA,
            pltpu.SemaphoreType.DMA,
            pltpu.SemaphoreType.DMA,
            pltpu.SemaphoreType.DMA,
        ],
        compiler_params=pltpu.CompilerParams(needs_layout_passes=False),
    )(tflat, wtab, ii16, zeros)
    return out.reshape(n_, c_, h_, w_)


# speedup vs baseline: 6.3375x; 6.3375x over previous
import jax
import jax.numpy as jnp
from jax.experimental import pallas as pl
from jax.experimental.pallas import tpu as pltpu

_CB = 7


def _nll2d_bwd_body(scal_ref, ii_ref, weight_ref, target_ref, out_ref):
    cb = pl.program_id(1)
    nclass = pl.num_programs(1) * _CB

    tgt = target_ref[0]
    tc = jnp.clip(tgt, 0, nclass - 1)
    tnorm = jnp.where(tgt == ii_ref[0], nclass, tc)
    for j in range(_CB):
        c = cb * _CB + j
        val = -scal_ref[0] * weight_ref[c]
        out_ref[0, j] = jnp.where(tnorm == c, val, 0.0)


def kernel(grad_output, x, target, weight, reduction, ignore_index, total_weight):
    n_, c_, h_, w_ = x.shape
    assert c_ % _CB == 0
    scal = jnp.where(reduction == 1, grad_output / total_weight, grad_output)
    scal = jnp.asarray(scal, x.dtype).reshape((1,))
    ii = jnp.asarray(ignore_index, jnp.int32).reshape((1,))
    weight = jnp.asarray(weight, x.dtype)

    out = pl.pallas_call(
        _nll2d_bwd_body,
        grid=(n_, c_ // _CB),
        in_specs=[
            pl.BlockSpec(memory_space=pltpu.SMEM),
            pl.BlockSpec(memory_space=pltpu.SMEM),
            pl.BlockSpec(memory_space=pltpu.SMEM),
            pl.BlockSpec((1, h_, w_), lambda n, c: (n, 0, 0)),
        ],
        out_specs=pl.BlockSpec((1, _CB, h_, w_), lambda n, c: (n, c, 0, 0)),
        out_shape=jax.ShapeDtypeStruct((n_, c_, h_, w_), x.dtype),
        compiler_params=pltpu.CompilerParams(
            dimension_semantics=("parallel", "arbitrary"),
        ),
    )(scal, ii, weight, target)
    return out


# speedup vs baseline: 6.3826x; 1.0071x over previous
import jax
import jax.numpy as jnp
from jax.experimental import pallas as pl
from jax.experimental.pallas import tpu as pltpu

_CB = 7
_NB = 2


def _nll2d_bwd_body(scal_ref, ii_ref, weight_ref, target_ref, out_ref):
    cb = pl.program_id(1)
    nclass = pl.num_programs(1) * _CB

    for b in range(_NB):
        tgt = target_ref[b]
        tc = jnp.clip(tgt, 0, nclass - 1)
        tnorm = jnp.where(tgt == ii_ref[0], nclass, tc)
        for j in range(_CB):
            c = cb * _CB + j
            val = -scal_ref[0] * weight_ref[c]
            out_ref[b, j] = jnp.where(tnorm == c, val, 0.0)


def kernel(grad_output, x, target, weight, reduction, ignore_index, total_weight):
    n_, c_, h_, w_ = x.shape
    assert c_ % _CB == 0 and n_ % _NB == 0
    scal = jnp.where(reduction == 1, grad_output / total_weight, grad_output)
    scal = jnp.asarray(scal, x.dtype).reshape((1,))
    ii = jnp.asarray(ignore_index, jnp.int32).reshape((1,))
    weight = jnp.asarray(weight, x.dtype)

    out = pl.pallas_call(
        _nll2d_bwd_body,
        grid=(n_ // _NB, c_ // _CB),
        in_specs=[
            pl.BlockSpec(memory_space=pltpu.SMEM),
            pl.BlockSpec(memory_space=pltpu.SMEM),
            pl.BlockSpec(memory_space=pltpu.SMEM),
            pl.BlockSpec((_NB, h_, w_), lambda n, c: (n, 0, 0)),
        ],
        out_specs=pl.BlockSpec((_NB, _CB, h_, w_), lambda n, c: (n, c, 0, 0)),
        out_shape=jax.ShapeDtypeStruct((n_, c_, h_, w_), x.dtype),
        compiler_params=pltpu.CompilerParams(
            dimension_semantics=("parallel", "arbitrary"),
        ),
    )(scal, ii, weight, target)
    return out
